# baseline (device time: 38776 ns/iter reference)
import jax
import jax.numpy as jnp
from jax import lax
from jax.experimental import pallas as pl
from jax.experimental.pallas import tpu as pltpu

N_DEV = 4
B_LOC = 2
SQ = 256
SKV = 256
HQ = 16
HQ_LOC = 4
DH = 64
D_MODEL = 512
HD_LOC = HQ_LOC * DH
WINDOW = 128


def _body(x_ref, wq_ref, kt_ref, vt_ref, wo_ref, out_ref,
          cwq, cwo, ctx_ref, swq, rwq, swo, rwo):
    my = lax.axis_index("i")

    bar = pltpu.get_barrier_semaphore()
    for r in range(1, N_DEV):
        peer = lax.rem(my + r, N_DEV)
        pl.semaphore_signal(bar, inc=1, device_id=(peer,),
                            device_id_type=pl.DeviceIdType.MESH)
    pl.semaphore_wait(bar, N_DEV - 1)

    sends = []
    for r in range(1, N_DEV):
        peer = lax.rem(my + r, N_DEV)
        rd_q = pltpu.make_async_remote_copy(
            src_ref=wq_ref, dst_ref=cwq.at[r - 1],
            send_sem=swq.at[r - 1], recv_sem=rwq.at[r - 1],
            device_id=(peer,), device_id_type=pl.DeviceIdType.MESH)
        rd_o = pltpu.make_async_remote_copy(
            src_ref=wo_ref, dst_ref=cwo.at[r - 1],
            send_sem=swo.at[r - 1], recv_sem=rwo.at[r - 1],
            device_id=(peer,), device_id_type=pl.DeviceIdType.MESH)
        rd_q.start()
        rd_o.start()
        sends.append((rd_q, rd_o))

    qi = lax.broadcasted_iota(jnp.int32, (SQ, SKV), 0)
    ki = lax.broadcasted_iota(jnp.int32, (SQ, SKV), 1)
    mask = jnp.abs(qi - ki) <= WINDOW

    def compute_chunk(origin, wq_c, wo_c, first):
        q = jnp.dot(x_ref[...], wq_c, preferred_element_type=jnp.float32)
        q = q * 0.125
        for b in range(B_LOC):
            kv_start = b * HQ + origin * HQ_LOC
            kb = kt_ref[pl.ds(kv_start, HQ_LOC)]
            vb = vt_ref[pl.ds(kv_start, HQ_LOC)]
            for h in range(HQ_LOC):
                qh = q[b * SQ:(b + 1) * SQ, h * DH:(h + 1) * DH]
                s = lax.dot_general(
                    qh, kb[h], (((1,), (1,)), ((), ())),
                    preferred_element_type=jnp.float32)
                s = jnp.where(mask, s, -1e9)
                s = s - jnp.max(s, axis=1, keepdims=True)
                w = jnp.exp(s)
                w = w / jnp.sum(w, axis=1, keepdims=True)
                ctx = jnp.dot(w, vb[h], preferred_element_type=jnp.float32)
                ctx_ref[b * SQ:(b + 1) * SQ, h * DH:(h + 1) * DH] = ctx
        part = jnp.dot(ctx_ref[...], wo_c, preferred_element_type=jnp.float32)
        if first:
            out_ref[...] = part
        else:
            out_ref[...] = out_ref[...] + part

    compute_chunk(my, wq_ref[...], wo_ref[...], first=True)
    for r in (1, 3, 2):
        rd_q, rd_o = sends[r - 1]
        rd_q.wait_recv()
        rd_o.wait_recv()
        origin = lax.rem(my - r + N_DEV, N_DEV)
        compute_chunk(origin, cwq[r - 1], cwo[r - 1], first=False)

    for rd_q, rd_o in sends:
        rd_q.wait_send()
        rd_o.wait_send()


def kernel(x, Wq, K_ext, V_ext, Wo):
    my = lax.axis_index("i")
    K_loc = lax.dynamic_slice_in_dim(K_ext, my * B_LOC, B_LOC, axis=0)
    V_loc = lax.dynamic_slice_in_dim(V_ext, my * B_LOC, B_LOC, axis=0)
    Kt = K_loc.transpose(0, 2, 1, 3).reshape(B_LOC * HQ, SKV, DH)
    Vt = V_loc.transpose(0, 2, 1, 3).reshape(B_LOC * HQ, SKV, DH)
    x2d = x.reshape(B_LOC * SQ, D_MODEL)

    out2d = pl.pallas_call(
        _body,
        out_shape=jax.ShapeDtypeStruct((B_LOC * SQ, D_MODEL), jnp.float32),
        in_specs=[pl.BlockSpec(memory_space=pltpu.VMEM)] * 5,
        out_specs=pl.BlockSpec(memory_space=pltpu.VMEM),
        scratch_shapes=[
            pltpu.VMEM((N_DEV - 1, D_MODEL, HD_LOC), jnp.float32),
            pltpu.VMEM((N_DEV - 1, HD_LOC, D_MODEL), jnp.float32),
            pltpu.VMEM((B_LOC * SQ, HD_LOC), jnp.float32),
            pltpu.SemaphoreType.DMA((N_DEV - 1,)),
            pltpu.SemaphoreType.DMA((N_DEV - 1,)),
            pltpu.SemaphoreType.DMA((N_DEV - 1,)),
            pltpu.SemaphoreType.DMA((N_DEV - 1,)),
        ],
        compiler_params=pltpu.CompilerParams(collective_id=0),
    )(x2d, Wq, Kt, Vt, Wo)
    return out2d.reshape(B_LOC, SQ, D_MODEL)


# device time: 28387 ns/iter; 1.3660x vs baseline; 1.3660x over previous
import jax
import jax.numpy as jnp
from jax import lax
from jax.experimental import pallas as pl
from jax.experimental.pallas import tpu as pltpu

N_DEV = 4
B_LOC = 2
SQ = 256
SKV = 256
HQ = 16
HQ_LOC = 4
DH = 64
D_MODEL = 512
HD_LOC = HQ_LOC * DH
WINDOW = 128


def _body(x_ref, wq_ref, kt_ref, vt_ref, wo_ref, out_ref,
          cwq, cwo, ctx_ref, swq, rwq, swo, rwo):
    my = lax.axis_index("i")

    bar = pltpu.get_barrier_semaphore()
    for r in range(1, N_DEV):
        peer = lax.rem(my + r, N_DEV)
        pl.semaphore_signal(bar, inc=1, device_id=(peer,),
                            device_id_type=pl.DeviceIdType.MESH)
    pl.semaphore_wait(bar, N_DEV - 1)

    sends = []
    for r in range(1, N_DEV):
        peer = lax.rem(my + r, N_DEV)
        rd_q = pltpu.make_async_remote_copy(
            src_ref=wq_ref, dst_ref=cwq.at[r - 1],
            send_sem=swq.at[r - 1], recv_sem=rwq.at[r - 1],
            device_id=(peer,), device_id_type=pl.DeviceIdType.MESH)
        rd_o = pltpu.make_async_remote_copy(
            src_ref=wo_ref, dst_ref=cwo.at[r - 1],
            send_sem=swo.at[r - 1], recv_sem=rwo.at[r - 1],
            device_id=(peer,), device_id_type=pl.DeviceIdType.MESH)
        rd_q.start()
        rd_o.start()
        sends.append((rd_q, rd_o))

    qi = lax.broadcasted_iota(jnp.int32, (SQ, SKV), 0)
    ki = lax.broadcasted_iota(jnp.int32, (SQ, SKV), 1)
    mask = jnp.abs(qi - ki) <= WINDOW

    def compute_chunk(origin, wq_c, wo_c, first):
        q = jnp.dot(x_ref[...], wq_c, preferred_element_type=jnp.float32)
        q = (q * 0.125).astype(jnp.bfloat16)
        for b in range(B_LOC):
            kv_start = b * HQ + origin * HQ_LOC
            kb = kt_ref[pl.ds(kv_start, HQ_LOC)]
            vb = vt_ref[pl.ds(kv_start, HQ_LOC)]
            for h in range(HQ_LOC):
                qh = q[b * SQ:(b + 1) * SQ, h * DH:(h + 1) * DH]
                s = lax.dot_general(
                    qh, kb[h], (((1,), (1,)), ((), ())),
                    preferred_element_type=jnp.float32)
                s = jnp.where(mask, s, -1e9)
                s = s - jnp.max(s, axis=1, keepdims=True)
                w = jnp.exp(s)
                w = (w / jnp.sum(w, axis=1, keepdims=True)).astype(jnp.bfloat16)
                ctx = jnp.dot(w, vb[h], preferred_element_type=jnp.float32)
                ctx_ref[b * SQ:(b + 1) * SQ, h * DH:(h + 1) * DH] = (
                    ctx.astype(jnp.bfloat16))
        part = jnp.dot(ctx_ref[...], wo_c, preferred_element_type=jnp.float32)
        if first:
            out_ref[...] = part
        else:
            out_ref[...] = out_ref[...] + part

    compute_chunk(my, wq_ref[...], wo_ref[...], first=True)
    for r in (1, 3, 2):
        rd_q, rd_o = sends[r - 1]
        rd_q.wait_recv()
        rd_o.wait_recv()
        origin = lax.rem(my - r + N_DEV, N_DEV)
        compute_chunk(origin, cwq[r - 1], cwo[r - 1], first=False)

    for rd_q, rd_o in sends:
        rd_q.wait_send()
        rd_o.wait_send()


def kernel(x, Wq, K_ext, V_ext, Wo):
    my = lax.axis_index("i")
    K_loc = lax.dynamic_slice_in_dim(K_ext, my * B_LOC, B_LOC, axis=0)
    V_loc = lax.dynamic_slice_in_dim(V_ext, my * B_LOC, B_LOC, axis=0)
    Kt = K_loc.astype(jnp.bfloat16).transpose(0, 2, 1, 3).reshape(
        B_LOC * HQ, SKV, DH)
    Vt = V_loc.astype(jnp.bfloat16).transpose(0, 2, 1, 3).reshape(
        B_LOC * HQ, SKV, DH)
    x2d = x.reshape(B_LOC * SQ, D_MODEL).astype(jnp.bfloat16)
    Wqb = Wq.astype(jnp.bfloat16)
    Wob = Wo.astype(jnp.bfloat16)

    out2d = pl.pallas_call(
        _body,
        out_shape=jax.ShapeDtypeStruct((B_LOC * SQ, D_MODEL), jnp.float32),
        in_specs=[pl.BlockSpec(memory_space=pltpu.VMEM)] * 5,
        out_specs=pl.BlockSpec(memory_space=pltpu.VMEM),
        scratch_shapes=[
            pltpu.VMEM((N_DEV - 1, D_MODEL, HD_LOC), jnp.bfloat16),
            pltpu.VMEM((N_DEV - 1, HD_LOC, D_MODEL), jnp.bfloat16),
            pltpu.VMEM((B_LOC * SQ, HD_LOC), jnp.bfloat16),
            pltpu.SemaphoreType.DMA((N_DEV - 1,)),
            pltpu.SemaphoreType.DMA((N_DEV - 1,)),
            pltpu.SemaphoreType.DMA((N_DEV - 1,)),
            pltpu.SemaphoreType.DMA((N_DEV - 1,)),
        ],
        compiler_params=pltpu.CompilerParams(collective_id=0),
    )(x2d, Wqb, Kt, Vt, Wob)
    return out2d.reshape(B_LOC, SQ, D_MODEL)
